# baseline (speedup 1.0000x reference)
import functools

import jax
import jax.numpy as jnp
from jax import lax
from jax.experimental import pallas as pl
from jax.experimental.pallas import tpu as pltpu
from jax.experimental.pallas import tpu_sc as plsc

N = 10000
NP = 10240
E = 320000
EP = 327680
D = 128
F2 = 64
C = 40
ECH = EP // 128

_NC, _NS = 2, 16


def _splat16(val_ref, idxs):
    return plsc.load_gather(val_ref, [jnp.full((16,), i, jnp.int32) for i in idxs])


def _front_body(ex_ref, w1_ref, b1_ref, w2_ref, b2_ref, x_ref, wc1_ref,
                ew_ref, x1_ref):
    e = jnp.dot(ex_ref[...], w1_ref[...], preferred_element_type=jnp.float32)
    e = jnp.maximum(e + b1_ref[...], 0.0)
    o = jnp.dot(e, w2_ref[...], preferred_element_type=jnp.float32) + b2_ref[...]
    ew_ref[...] = jax.nn.sigmoid(o)
    r = jnp.dot(x_ref[...], wc1_ref[...], preferred_element_type=jnp.float32)
    x1_ref[0] = r[:, 0:64]
    x1_ref[1] = r[:, 64:128]


def _front(edge_x, W1, b1, W2, b2, xp, Wc1):
    be = E // 20
    bn = NP // 20
    return pl.pallas_call(
        _front_body,
        grid=(20,),
        in_specs=[
            pl.BlockSpec((be, 16), lambda i: (i, 0)),
            pl.BlockSpec((16, 16), lambda i: (0, 0)),
            pl.BlockSpec((1, 16), lambda i: (0, 0)),
            pl.BlockSpec((16, 1), lambda i: (0, 0)),
            pl.BlockSpec((1, 1), lambda i: (0, 0)),
            pl.BlockSpec((bn, D), lambda i: (i, 0)),
            pl.BlockSpec((D, D), lambda i: (0, 0)),
        ],
        out_specs=[
            pl.BlockSpec((be, 1), lambda i: (i, 0)),
            pl.BlockSpec((2, bn, 64), lambda i: (0, i, 0)),
        ],
        out_shape=[
            jax.ShapeDtypeStruct((E, 1), jnp.float32),
            jax.ShapeDtypeStruct((2, NP, 64), jnp.float32),
        ],
    )(edge_x, W1, b1.reshape(1, 16), W2, b2.reshape(1, 1), xp, Wc1)


def _xw1_body(x_ref, w_ref, o_ref):
    o_ref[...] = jnp.dot(x_ref[...], w_ref[0],
                         preferred_element_type=jnp.float32)[None]


def _xw1(xp, Wc1s):
    bn = 1024
    return pl.pallas_call(
        _xw1_body,
        grid=(NP // bn, 2),
        in_specs=[
            pl.BlockSpec((bn, D), lambda i, c: (i, 0)),
            pl.BlockSpec((1, D, D // 2), lambda i, c: (c, 0, 0)),
        ],
        out_specs=pl.BlockSpec((1, bn, D // 2), lambda i, c: (c, i, 0)),
        out_shape=jax.ShapeDtypeStruct((2, NP, D // 2), jnp.float32),
    )(xp, Wc1s)


def _scale1_body(da_ref, db_ref, x_ref, dinv_ref, xs_ref):
    d = lax.rsqrt(1.0 + da_ref[0] + db_ref[0])
    dinv_ref[...] = d
    xs_ref[0] = x_ref[0] * d


def _scale1(degAB3, X1):
    bn = 1024
    return pl.pallas_call(
        _scale1_body,
        grid=(NP // bn, 2),
        in_specs=[
            pl.BlockSpec((1, bn, 1), lambda i, c: (0, i, 0)),
            pl.BlockSpec((1, bn, 1), lambda i, c: (1, i, 0)),
            pl.BlockSpec((1, bn, D // 2), lambda i, c: (c, i, 0)),
        ],
        out_specs=[
            pl.BlockSpec((bn, 1), lambda i, c: (i, 0)),
            pl.BlockSpec((1, bn, D // 2), lambda i, c: (c, i, 0)),
        ],
        out_shape=[
            jax.ShapeDtypeStruct((NP, 1), jnp.float32),
            jax.ShapeDtypeStruct((2, NP, D // 2), jnp.float32),
        ],
    )(degAB3, degAB3, X1)


def _mm2_body(ha_ref, hb_ref, w_ref, dinv_ref, xs_ref):
    x2 = (jnp.dot(ha_ref[0], w_ref[0:64, :], preferred_element_type=jnp.float32)
          + jnp.dot(hb_ref[0], w_ref[64:128, :],
                    preferred_element_type=jnp.float32))
    xs_ref[...] = x2 * dinv_ref[...]


def _mm2(h1, Wc2p, dinv):
    bn = 1024
    return pl.pallas_call(
        _mm2_body,
        grid=(NP // bn,),
        in_specs=[
            pl.BlockSpec((1, bn, 64), lambda i: (0, i, 0)),
            pl.BlockSpec((1, bn, 64), lambda i: (1, i, 0)),
            pl.BlockSpec((D, F2), lambda i: (0, 0)),
            pl.BlockSpec((bn, 1), lambda i: (i, 0)),
        ],
        out_specs=pl.BlockSpec((bn, F2), lambda i: (i, 0)),
        out_shape=jax.ShapeDtypeStruct((NP, F2), jnp.float32),
    )(h1, h1, Wc2p, dinv)


def _sc_mesh():
    return plsc.VectorSubcoreMesh(core_axis_name="c", subcore_axis_name="s")


_SC_PARAMS = pltpu.CompilerParams(needs_layout_passes=False,
                                  use_tc_tiling_on_sc=False)


def _deg_body(col_hbm, ew_hbm, zeros_hbm, deg_out,
              deg_sh, cbuf, ebuf):
    c = lax.axis_index("c")
    s = lax.axis_index("s")
    w = c * _NS + s
    n0 = s * (NP // _NS)
    pltpu.sync_copy(zeros_hbm.at[pl.ds(n0, NP // _NS)],
                    deg_sh.at[pl.ds(n0, NP // _NS)])
    plsc.subcore_barrier()
    nrow = ECH // (_NC * _NS)

    def chunk(j, carry):
        pltpu.sync_copy(ebuf.at[j], deg_sh.at[cbuf.at[j]], add=True)
        return carry

    e0 = w * nrow
    pltpu.sync_copy(col_hbm.at[pl.ds(e0, nrow)], cbuf)
    pltpu.sync_copy(ew_hbm.at[pl.ds(e0, nrow)], ebuf)
    lax.fori_loop(0, nrow, chunk, 0)
    plsc.subcore_barrier()
    pltpu.sync_copy(deg_sh.at[pl.ds(n0, NP // _NS)],
                    deg_out.at[pl.ds(c * NP + n0, NP // _NS)])


def _deg_kernel(colp2, ewp2, zeros_np):
    k = pl.kernel(
        _deg_body,
        out_type=jax.ShapeDtypeStruct((2 * NP,), jnp.float32),
        mesh=_sc_mesh(),
        compiler_params=_SC_PARAMS,
        scratch_types=[
            pltpu.VMEM_SHARED((NP,), jnp.float32),
            pltpu.VMEM((ECH // 32, 128), jnp.int32),
            pltpu.VMEM((ECH // 32, 128), jnp.float32),
        ],
    )
    return k(colp2, ewp2, zeros_np)


def _spmm1_body(row_hbm, col_hbm, ew_hbm, x_hbm, b_hbm,
                out_hbm, dinv_out, deg_sh, xs_sh, acc_sh, rbuf, cbuf, ebuf,
                gbuf2, obuf, degbuf, dbuf, bbuf, xbuf, gsem0, gsem1):
    F = 64
    nv = F // 16
    c = lax.axis_index("c")
    s = lax.axis_index("s")
    n0 = s * (NP // _NS)
    nblk = NP // _NS
    nch = ECH // _NS
    e0 = s * nch

    def zset(k, carry):
        degbuf[pl.ds(k * 16, 16)] = jnp.zeros((16,), jnp.float32)
        return carry

    lax.fori_loop(0, nblk // 16, zset, 0)
    pltpu.sync_copy(degbuf, deg_sh.at[pl.ds(n0, nblk)])
    pltpu.sync_copy(b_hbm.at[c], bbuf)
    plsc.subcore_barrier()

    def dgroup(g, carry):
        gg = e0 + g * _GR
        pltpu.sync_copy(col_hbm.at[pl.ds(gg, _GR)], cbuf)
        pltpu.sync_copy(ew_hbm.at[pl.ds(gg, _GR)], ebuf)

        def dchunk(j, carry2):
            pltpu.sync_copy(ebuf.at[j], deg_sh.at[cbuf.at[j]], add=True)
            return carry2

        lax.fori_loop(0, _GR, dchunk, 0)
        return carry

    lax.fori_loop(0, nch // _GR, dgroup, 0)
    plsc.subcore_barrier()

    pltpu.sync_copy(deg_sh.at[pl.ds(n0, nblk)], degbuf)

    def newton(k, carry):
        x = degbuf[pl.ds(k * 16, 16)] + 1.0
        i = plsc.bitcast(x, jnp.int32)
        i = 0x5F3759DF - lax.shift_right_logical(i, 1)
        y = plsc.bitcast(i, jnp.float32)
        for _ in range(3):
            y = y * (1.5 - 0.5 * x * y * y)
        dbuf[pl.ds(k * 16, 16)] = y
        return carry

    lax.fori_loop(0, nblk // 16, newton, 0)

    @pl.when(c == 0)
    def _():
        pltpu.sync_copy(dbuf, dinv_out.at[pl.ds(n0, nblk)])

    def xstage(t, carry):
        b0 = t * _XB
        pltpu.sync_copy(x_hbm.at[c, pl.ds(n0 + b0, _XB)], xbuf)

        def xscale(i, cc):
            sd = _splat16(dbuf, (b0 + i,))
            for f in range(nv):
                v = xbuf[i, pl.ds(f * 16, 16)]
                xbuf[i, pl.ds(f * 16, 16)] = v * sd
            return cc

        lax.fori_loop(0, _XB, xscale, 0)
        pltpu.sync_copy(xbuf, xs_sh.at[pl.ds(n0 + b0, _XB)])
        pltpu.sync_copy(xbuf, acc_sh.at[pl.ds(n0 + b0, _XB)])
        return carry

    lax.fori_loop(0, nblk // _XB, xstage, 0)
    plsc.subcore_barrier()

    gsems = (gsem0, gsem1)

    def start_gather(j, b):
        pltpu.async_copy(xs_sh.at[rbuf.at[j]], gbuf2.at[b], gsems[b])

    def wait_gather(j, b):
        pltpu.make_async_copy(xs_sh.at[rbuf.at[j]], gbuf2.at[b],
                              gsems[b]).wait()

    def chunk(j, b):
        wait_gather(j, b)

        def edge(i8, cc):
            for di in range(8):
                i = i8 * 8 + di
                sp = _splat16(ebuf, (j, i))
                for f in range(nv):
                    v = gbuf2[b, i, pl.ds(f * 16, 16)]
                    gbuf2[b, i, pl.ds(f * 16, 16)] = v * sp
            return cc

        lax.fori_loop(0, 16, edge, 0)
        pltpu.sync_copy(gbuf2.at[b], acc_sh.at[cbuf.at[j]], add=True)

    def group(g, carry):
        gg = g * _GR
        pltpu.sync_copy(row_hbm.at[pl.ds(e0 + gg, _GR)], rbuf)
        pltpu.sync_copy(col_hbm.at[pl.ds(e0 + gg, _GR)], cbuf)
        pltpu.sync_copy(ew_hbm.at[pl.ds(e0 + gg, _GR)], ebuf)
        start_gather(0, 0)

        def pair(j2, carry2):
            j0 = j2 * 2
            start_gather(j0 + 1, 1)
            chunk(j0, 0)

            @pl.when(j2 < _GR // 2 - 1)
            def _():
                start_gather(j0 + 2, 0)
            chunk(j0 + 1, 1)
            return carry2

        lax.fori_loop(0, _GR // 2, pair, 0)
        return carry

    lax.fori_loop(0, nch // _GR, group, 0)
    plsc.subcore_barrier()

    def finblk(t, carry):
        b0 = t * _FB
        pltpu.sync_copy(acc_sh.at[pl.ds(n0 + b0, _FB)], obuf)

        def fin(i, cc):
            sd = _splat16(dbuf, (b0 + i,))
            for f in range(nv):
                v = obuf[i, pl.ds(f * 16, 16)]
                b = bbuf[0, pl.ds(f * 16, 16)]
                v = jnp.maximum(v * sd + b, 0.0)
                obuf[i, pl.ds(f * 16, 16)] = v
            return cc

        lax.fori_loop(0, _FB, fin, 0)
        pltpu.sync_copy(obuf, out_hbm.at[c, pl.ds(n0 + b0, _FB)])
        return carry

    lax.fori_loop(0, nblk // _FB, finblk, 0)


_XB = 160


def _spmm1_kernel(rowp2, colp2, ewp2, X1, bias):
    k = pl.kernel(
        _spmm1_body,
        out_type=[jax.ShapeDtypeStruct((2, NP, 64), jnp.float32),
                  jax.ShapeDtypeStruct((NP,), jnp.float32)],
        mesh=_sc_mesh(),
        compiler_params=_SC_PARAMS,
        scratch_types=[
            pltpu.VMEM_SHARED((NP,), jnp.float32),
            pltpu.VMEM_SHARED((NP, 64), jnp.float32),
            pltpu.VMEM_SHARED((NP, 64), jnp.float32),
            pltpu.VMEM((_GR, 128), jnp.int32),
            pltpu.VMEM((_GR, 128), jnp.int32),
            pltpu.VMEM((_GR, 128), jnp.float32),
            pltpu.VMEM((2, 128, 64), jnp.float32),
            pltpu.VMEM((_FB, 64), jnp.float32),
            pltpu.VMEM((NP // _NS,), jnp.float32),
            pltpu.VMEM((NP // _NS,), jnp.float32),
            pltpu.VMEM((1, 64), jnp.float32),
            pltpu.VMEM((_XB, 64), jnp.float32),
            pltpu.SemaphoreType.DMA,
            pltpu.SemaphoreType.DMA,
        ],
    )
    return k(rowp2, colp2, ewp2, X1, bias)


def _spmm_body(F, row_hbm, col_hbm, ew_hbm, xs_hbm, dinv_hbm, b_hbm,
               out_hbm, xs_sh, acc_sh, rbuf, cbuf, ebuf, gbuf2, obuf,
               dbuf, bbuf, gsem0, gsem1):
    nv = F // 16
    c = lax.axis_index("c")
    s = lax.axis_index("s")
    n0 = s * (NP // _NS)
    nblk = NP // _NS
    pltpu.sync_copy(xs_hbm.at[c, pl.ds(n0, nblk)], xs_sh.at[pl.ds(n0, nblk)])
    pltpu.sync_copy(xs_hbm.at[c, pl.ds(n0, nblk)], acc_sh.at[pl.ds(n0, nblk)])
    pltpu.sync_copy(dinv_hbm.at[pl.ds(n0, nblk)], dbuf)
    pltpu.sync_copy(b_hbm.at[c], bbuf)
    plsc.subcore_barrier()

    nch = ECH // _NS
    e0 = s * nch
    gsems = (gsem0, gsem1)

    def start_gather(j, b):
        pltpu.async_copy(xs_sh.at[rbuf.at[j]], gbuf2.at[b], gsems[b])

    def wait_gather(j, b):
        pltpu.make_async_copy(xs_sh.at[rbuf.at[j]], gbuf2.at[b],
                              gsems[b]).wait()

    def chunk(j, b):
        wait_gather(j, b)

        def edge(i8, cc):
            for di in range(8):
                i = i8 * 8 + di
                sp = _splat16(ebuf, (j, i))
                for f in range(nv):
                    v = gbuf2[b, i, pl.ds(f * 16, 16)]
                    gbuf2[b, i, pl.ds(f * 16, 16)] = v * sp
            return cc

        lax.fori_loop(0, 16, edge, 0)
        pltpu.sync_copy(gbuf2.at[b], acc_sh.at[cbuf.at[j]], add=True)

    def group(g, carry):
        gg = g * _GR
        pltpu.sync_copy(row_hbm.at[pl.ds(e0 + gg, _GR)], rbuf)
        pltpu.sync_copy(col_hbm.at[pl.ds(e0 + gg, _GR)], cbuf)
        pltpu.sync_copy(ew_hbm.at[pl.ds(e0 + gg, _GR)], ebuf)
        start_gather(0, 0)

        def pair(j2, carry2):
            j0 = j2 * 2
            start_gather(j0 + 1, 1)
            chunk(j0, 0)

            @pl.when(j2 < _GR // 2 - 1)
            def _():
                start_gather(j0 + 2, 0)
            chunk(j0 + 1, 1)
            return carry2

        lax.fori_loop(0, _GR // 2, pair, 0)
        return carry

    lax.fori_loop(0, nch // _GR, group, 0)
    plsc.subcore_barrier()

    def finblk(t, carry):
        b0 = t * _FB
        pltpu.sync_copy(acc_sh.at[pl.ds(n0 + b0, _FB)], obuf)

        def fin(i, cc):
            sd = _splat16(dbuf, (b0 + i,))
            for f in range(nv):
                v = obuf[i, pl.ds(f * 16, 16)]
                b = bbuf[0, pl.ds(f * 16, 16)]
                v = v * sd + b
                if F == 64:
                    v = jnp.maximum(v, 0.0)
                obuf[i, pl.ds(f * 16, 16)] = v
            return cc

        lax.fori_loop(0, _FB, fin, 0)
        pltpu.sync_copy(obuf, out_hbm.at[c, pl.ds(n0 + b0, _FB)])
        return carry

    lax.fori_loop(0, nblk // _FB, finblk, 0)


_GR = 16
_FB = 64


def _spmm_kernel(F, rowp2, colp2, ewp2, Xs, dinv, bias):
    k = pl.kernel(
        functools.partial(_spmm_body, F),
        out_type=jax.ShapeDtypeStruct((2, NP, F), jnp.float32),
        mesh=_sc_mesh(),
        compiler_params=_SC_PARAMS,
        scratch_types=[
            pltpu.VMEM_SHARED((NP, F), jnp.float32),
            pltpu.VMEM_SHARED((NP, F), jnp.float32),
            pltpu.VMEM((_GR, 128), jnp.int32),
            pltpu.VMEM((_GR, 128), jnp.int32),
            pltpu.VMEM((_GR, 128), jnp.float32),
            pltpu.VMEM((2, 128, F), jnp.float32),
            pltpu.VMEM((_FB, F), jnp.float32),
            pltpu.VMEM((NP // _NS,), jnp.float32),
            pltpu.VMEM((1, F), jnp.float32),
            pltpu.SemaphoreType.DMA,
            pltpu.SemaphoreType.DMA,
        ],
    )
    return k(rowp2, colp2, ewp2, Xs, dinv, bias)


def kernel(x, edge_index, edge_x, W1, b1, W2, b2, Wc1, bc1, Wc2, bc2):
    xp = jnp.pad(x, ((0, NP - N), (0, 0)))
    rowp = jnp.pad(edge_index[0], (0, EP - E)).reshape(ECH, 128)
    colp = jnp.pad(edge_index[1], (0, EP - E)).reshape(ECH, 128)
    Wc2p = jnp.pad(Wc2, ((0, 0), (0, F2 - C)))
    bc2p = jnp.pad(bc2, (0, F2 - C))

    ew, X1 = _front(edge_x, W1, b1, W2, b2, xp, Wc1)
    ewp = jnp.pad(ew.reshape(-1), (0, EP - E)).reshape(ECH, 128)

    h1, dinv = _spmm1_kernel(rowp, colp, ewp, X1,
                             bc1.reshape(2, 1, 64))

    Xs2v = _mm2(h1, Wc2p, dinv.reshape(NP, 1))
    Xs2 = Xs2v.reshape(NP, 2, 32).transpose(1, 0, 2)

    out2 = _spmm_kernel(32, rowp, colp, ewp, Xs2, dinv,
                        bc2p.reshape(2, 1, 32))
    out = out2.transpose(1, 0, 2).reshape(NP, F2)
    return out[:N, :C]


# speedup vs baseline: 1.0416x; 1.0416x over previous
import functools

import jax
import jax.numpy as jnp
from jax import lax
from jax.experimental import pallas as pl
from jax.experimental.pallas import tpu as pltpu
from jax.experimental.pallas import tpu_sc as plsc

N = 10000
NP = 10240
E = 320000
EP = 327680
D = 128
F2 = 64
C = 40
ECH = EP // 128

_NC, _NS = 2, 16


def _splat16(val_ref, idxs):
    return plsc.load_gather(val_ref, [jnp.full((16,), i, jnp.int32) for i in idxs])


def _front_body(ex_ref, w1_ref, b1_ref, w2_ref, b2_ref, x_ref, wc1_ref,
                ew_ref, x1_ref):
    e = jnp.dot(ex_ref[...], w1_ref[...], preferred_element_type=jnp.float32)
    e = jnp.maximum(e + b1_ref[...], 0.0)
    o = jnp.dot(e, w2_ref[...], preferred_element_type=jnp.float32) + b2_ref[...]
    ew_ref[...] = jax.nn.sigmoid(o)
    r = jnp.dot(x_ref[...], wc1_ref[...], preferred_element_type=jnp.float32)
    x1_ref[0] = r[:, 0:64]
    x1_ref[1] = r[:, 64:128]


def _front(edge_x, W1, b1, W2, b2, xp, Wc1):
    be = E // 20
    bn = NP // 20
    return pl.pallas_call(
        _front_body,
        grid=(20,),
        in_specs=[
            pl.BlockSpec((be, 16), lambda i: (i, 0)),
            pl.BlockSpec((16, 16), lambda i: (0, 0)),
            pl.BlockSpec((1, 16), lambda i: (0, 0)),
            pl.BlockSpec((16, 1), lambda i: (0, 0)),
            pl.BlockSpec((1, 1), lambda i: (0, 0)),
            pl.BlockSpec((bn, D), lambda i: (i, 0)),
            pl.BlockSpec((D, D), lambda i: (0, 0)),
        ],
        out_specs=[
            pl.BlockSpec((be, 1), lambda i: (i, 0)),
            pl.BlockSpec((2, bn, 64), lambda i: (0, i, 0)),
        ],
        out_shape=[
            jax.ShapeDtypeStruct((E, 1), jnp.float32),
            jax.ShapeDtypeStruct((2, NP, 64), jnp.float32),
        ],
    )(edge_x, W1, b1.reshape(1, 16), W2, b2.reshape(1, 1), xp, Wc1)


def _xw1_body(x_ref, w_ref, o_ref):
    o_ref[...] = jnp.dot(x_ref[...], w_ref[0],
                         preferred_element_type=jnp.float32)[None]


def _xw1(xp, Wc1s):
    bn = 1024
    return pl.pallas_call(
        _xw1_body,
        grid=(NP // bn, 2),
        in_specs=[
            pl.BlockSpec((bn, D), lambda i, c: (i, 0)),
            pl.BlockSpec((1, D, D // 2), lambda i, c: (c, 0, 0)),
        ],
        out_specs=pl.BlockSpec((1, bn, D // 2), lambda i, c: (c, i, 0)),
        out_shape=jax.ShapeDtypeStruct((2, NP, D // 2), jnp.float32),
    )(xp, Wc1s)


def _scale1_body(da_ref, db_ref, x_ref, dinv_ref, xs_ref):
    d = lax.rsqrt(1.0 + da_ref[0] + db_ref[0])
    dinv_ref[...] = d
    xs_ref[0] = x_ref[0] * d


def _scale1(degAB3, X1):
    bn = 1024
    return pl.pallas_call(
        _scale1_body,
        grid=(NP // bn, 2),
        in_specs=[
            pl.BlockSpec((1, bn, 1), lambda i, c: (0, i, 0)),
            pl.BlockSpec((1, bn, 1), lambda i, c: (1, i, 0)),
            pl.BlockSpec((1, bn, D // 2), lambda i, c: (c, i, 0)),
        ],
        out_specs=[
            pl.BlockSpec((bn, 1), lambda i, c: (i, 0)),
            pl.BlockSpec((1, bn, D // 2), lambda i, c: (c, i, 0)),
        ],
        out_shape=[
            jax.ShapeDtypeStruct((NP, 1), jnp.float32),
            jax.ShapeDtypeStruct((2, NP, D // 2), jnp.float32),
        ],
    )(degAB3, degAB3, X1)


def _mm2_body(ha_ref, hb_ref, w_ref, dinv_ref, xs_ref):
    x2 = (jnp.dot(ha_ref[0], w_ref[0:64, :], preferred_element_type=jnp.float32)
          + jnp.dot(hb_ref[0], w_ref[64:128, :],
                    preferred_element_type=jnp.float32))
    xs_ref[...] = x2 * dinv_ref[...]


def _mm2(h1, Wc2p, dinv):
    bn = 1024
    return pl.pallas_call(
        _mm2_body,
        grid=(NP // bn,),
        in_specs=[
            pl.BlockSpec((1, bn, 64), lambda i: (0, i, 0)),
            pl.BlockSpec((1, bn, 64), lambda i: (1, i, 0)),
            pl.BlockSpec((D, F2), lambda i: (0, 0)),
            pl.BlockSpec((bn, 1), lambda i: (i, 0)),
        ],
        out_specs=pl.BlockSpec((bn, F2), lambda i: (i, 0)),
        out_shape=jax.ShapeDtypeStruct((NP, F2), jnp.float32),
    )(h1, h1, Wc2p, dinv)


def _sc_mesh():
    return plsc.VectorSubcoreMesh(core_axis_name="c", subcore_axis_name="s")


_SC_PARAMS = pltpu.CompilerParams(needs_layout_passes=False,
                                  use_tc_tiling_on_sc=False)


def _deg_body(col_hbm, ew_hbm, zeros_hbm, deg_out,
              deg_sh, cbuf, ebuf):
    c = lax.axis_index("c")
    s = lax.axis_index("s")
    w = c * _NS + s
    n0 = s * (NP // _NS)
    pltpu.sync_copy(zeros_hbm.at[pl.ds(n0, NP // _NS)],
                    deg_sh.at[pl.ds(n0, NP // _NS)])
    plsc.subcore_barrier()
    nrow = ECH // (_NC * _NS)

    def chunk(j, carry):
        pltpu.sync_copy(ebuf.at[j], deg_sh.at[cbuf.at[j]], add=True)
        return carry

    e0 = w * nrow
    pltpu.sync_copy(col_hbm.at[pl.ds(e0, nrow)], cbuf)
    pltpu.sync_copy(ew_hbm.at[pl.ds(e0, nrow)], ebuf)
    lax.fori_loop(0, nrow, chunk, 0)
    plsc.subcore_barrier()
    pltpu.sync_copy(deg_sh.at[pl.ds(n0, NP // _NS)],
                    deg_out.at[pl.ds(c * NP + n0, NP // _NS)])


def _deg_kernel(colp2, ewp2, zeros_np):
    k = pl.kernel(
        _deg_body,
        out_type=jax.ShapeDtypeStruct((2 * NP,), jnp.float32),
        mesh=_sc_mesh(),
        compiler_params=_SC_PARAMS,
        scratch_types=[
            pltpu.VMEM_SHARED((NP,), jnp.float32),
            pltpu.VMEM((ECH // 32, 128), jnp.int32),
            pltpu.VMEM((ECH // 32, 128), jnp.float32),
        ],
    )
    return k(colp2, ewp2, zeros_np)


def _spmm1_body(row_hbm, col_hbm, ew_hbm, x_hbm, b_hbm,
                out_hbm, dinv_out, deg_sh, xs_sh, acc_sh, rbuf, cbuf, ebuf,
                gbuf2, obuf, degbuf, dbuf, bbuf, xbuf, gsem0, gsem1):
    F = 64
    nv = F // 16
    c = lax.axis_index("c")
    s = lax.axis_index("s")
    n0 = s * (NP // _NS)
    nblk = NP // _NS
    nch = ECH // _NS
    e0 = s * nch

    def zset(k, carry):
        degbuf[pl.ds(k * 16, 16)] = jnp.zeros((16,), jnp.float32)
        return carry

    lax.fori_loop(0, nblk // 16, zset, 0)
    pltpu.sync_copy(degbuf, deg_sh.at[pl.ds(n0, nblk)])
    pltpu.sync_copy(b_hbm.at[c], bbuf)
    plsc.subcore_barrier()

    def dgroup(g, carry):
        gg = e0 + g * _GR
        pltpu.sync_copy(col_hbm.at[pl.ds(gg, _GR)], cbuf)
        pltpu.sync_copy(ew_hbm.at[pl.ds(gg, _GR)], ebuf)

        def dfire(j, carry2):
            pltpu.async_copy(ebuf.at[j], deg_sh.at[cbuf.at[j]], gsem0,
                             add=True)
            return carry2

        def ddrain(j, carry2):
            pltpu.make_async_copy(ebuf.at[j], deg_sh.at[cbuf.at[j]],
                                  gsem0).wait()
            return carry2

        lax.fori_loop(0, _GR, dfire, 0)
        lax.fori_loop(0, _GR, ddrain, 0)
        return carry

    lax.fori_loop(0, nch // _GR, dgroup, 0)
    plsc.subcore_barrier()

    pltpu.sync_copy(deg_sh.at[pl.ds(n0, nblk)], degbuf)

    def newton(k, carry):
        x = degbuf[pl.ds(k * 16, 16)] + 1.0
        i = plsc.bitcast(x, jnp.int32)
        i = 0x5F3759DF - lax.shift_right_logical(i, 1)
        y = plsc.bitcast(i, jnp.float32)
        for _ in range(3):
            y = y * (1.5 - 0.5 * x * y * y)
        dbuf[pl.ds(k * 16, 16)] = y
        return carry

    lax.fori_loop(0, nblk // 16, newton, 0)

    @pl.when(c == 0)
    def _():
        pltpu.sync_copy(dbuf, dinv_out.at[pl.ds(n0, nblk)])

    def xstage(t, carry):
        b0 = t * _XB
        pltpu.sync_copy(x_hbm.at[c, pl.ds(n0 + b0, _XB)], xbuf)

        def xscale(i, cc):
            sd = _splat16(dbuf, (b0 + i,))
            for f in range(nv):
                v = xbuf[i, pl.ds(f * 16, 16)]
                xbuf[i, pl.ds(f * 16, 16)] = v * sd
            return cc

        lax.fori_loop(0, _XB, xscale, 0)
        pltpu.sync_copy(xbuf, xs_sh.at[pl.ds(n0 + b0, _XB)])
        pltpu.sync_copy(xbuf, acc_sh.at[pl.ds(n0 + b0, _XB)])
        return carry

    lax.fori_loop(0, nblk // _XB, xstage, 0)
    plsc.subcore_barrier()

    gsems = (gsem0, gsem1)

    def start_gather(j, b):
        pltpu.async_copy(xs_sh.at[rbuf.at[j]], gbuf2.at[b], gsems[b])

    def wait_gather(j, b):
        pltpu.make_async_copy(xs_sh.at[rbuf.at[j]], gbuf2.at[b],
                              gsems[b]).wait()

    def chunk(j, b):
        wait_gather(j, b)

        def edge(i8, cc):
            for di in range(8):
                i = i8 * 8 + di
                sp = _splat16(ebuf, (j, i))
                for f in range(nv):
                    v = gbuf2[b, i, pl.ds(f * 16, 16)]
                    gbuf2[b, i, pl.ds(f * 16, 16)] = v * sp
            return cc

        lax.fori_loop(0, 16, edge, 0)
        pltpu.sync_copy(gbuf2.at[b], acc_sh.at[cbuf.at[j]], add=True)

    def group(g, carry):
        gg = g * _GR
        pltpu.sync_copy(row_hbm.at[pl.ds(e0 + gg, _GR)], rbuf)
        pltpu.sync_copy(col_hbm.at[pl.ds(e0 + gg, _GR)], cbuf)
        pltpu.sync_copy(ew_hbm.at[pl.ds(e0 + gg, _GR)], ebuf)
        start_gather(0, 0)

        def pair(j2, carry2):
            j0 = j2 * 2
            start_gather(j0 + 1, 1)
            chunk(j0, 0)

            @pl.when(j2 < _GR // 2 - 1)
            def _():
                start_gather(j0 + 2, 0)
            chunk(j0 + 1, 1)
            return carry2

        lax.fori_loop(0, _GR // 2, pair, 0)
        return carry

    lax.fori_loop(0, nch // _GR, group, 0)
    plsc.subcore_barrier()

    def finblk(t, carry):
        b0 = t * _FB
        pltpu.sync_copy(acc_sh.at[pl.ds(n0 + b0, _FB)], obuf)

        def fin(i, cc):
            sd = _splat16(dbuf, (b0 + i,))
            for f in range(nv):
                v = obuf[i, pl.ds(f * 16, 16)]
                b = bbuf[0, pl.ds(f * 16, 16)]
                v = jnp.maximum(v * sd + b, 0.0)
                obuf[i, pl.ds(f * 16, 16)] = v
            return cc

        lax.fori_loop(0, _FB, fin, 0)
        pltpu.sync_copy(obuf, out_hbm.at[c, pl.ds(n0 + b0, _FB)])
        return carry

    lax.fori_loop(0, nblk // _FB, finblk, 0)


_XB = 160


def _spmm1_kernel(rowp2, colp2, ewp2, X1, bias):
    k = pl.kernel(
        _spmm1_body,
        out_type=[jax.ShapeDtypeStruct((2, NP, 64), jnp.float32),
                  jax.ShapeDtypeStruct((NP,), jnp.float32)],
        mesh=_sc_mesh(),
        compiler_params=_SC_PARAMS,
        scratch_types=[
            pltpu.VMEM_SHARED((NP,), jnp.float32),
            pltpu.VMEM_SHARED((NP, 64), jnp.float32),
            pltpu.VMEM_SHARED((NP, 64), jnp.float32),
            pltpu.VMEM((_GR, 128), jnp.int32),
            pltpu.VMEM((_GR, 128), jnp.int32),
            pltpu.VMEM((_GR, 128), jnp.float32),
            pltpu.VMEM((2, 128, 64), jnp.float32),
            pltpu.VMEM((_FB, 64), jnp.float32),
            pltpu.VMEM((NP // _NS,), jnp.float32),
            pltpu.VMEM((NP // _NS,), jnp.float32),
            pltpu.VMEM((1, 64), jnp.float32),
            pltpu.VMEM((_XB, 64), jnp.float32),
            pltpu.SemaphoreType.DMA,
            pltpu.SemaphoreType.DMA,
        ],
    )
    return k(rowp2, colp2, ewp2, X1, bias)


def _spmm_body(F, row_hbm, col_hbm, ew_hbm, xs_hbm, dinv_hbm, b_hbm,
               out_hbm, xs_sh, acc_sh, rbuf, cbuf, ebuf, gbuf2, obuf,
               dbuf, bbuf, gsem0, gsem1):
    nv = F // 16
    c = lax.axis_index("c")
    s = lax.axis_index("s")
    n0 = s * (NP // _NS)
    nblk = NP // _NS
    pltpu.sync_copy(xs_hbm.at[c, pl.ds(n0, nblk)], xs_sh.at[pl.ds(n0, nblk)])
    pltpu.sync_copy(xs_hbm.at[c, pl.ds(n0, nblk)], acc_sh.at[pl.ds(n0, nblk)])
    pltpu.sync_copy(dinv_hbm.at[pl.ds(n0, nblk)], dbuf)
    pltpu.sync_copy(b_hbm.at[c], bbuf)
    plsc.subcore_barrier()

    nch = ECH // _NS
    e0 = s * nch
    gsems = (gsem0, gsem1)

    def start_gather(j, b):
        pltpu.async_copy(xs_sh.at[rbuf.at[j]], gbuf2.at[b], gsems[b])

    def wait_gather(j, b):
        pltpu.make_async_copy(xs_sh.at[rbuf.at[j]], gbuf2.at[b],
                              gsems[b]).wait()

    def chunk(j, b):
        wait_gather(j, b)

        def edge(i8, cc):
            for di in range(8):
                i = i8 * 8 + di
                sp = _splat16(ebuf, (j, i))
                for f in range(nv):
                    v = gbuf2[b, i, pl.ds(f * 16, 16)]
                    gbuf2[b, i, pl.ds(f * 16, 16)] = v * sp
            return cc

        lax.fori_loop(0, 16, edge, 0)
        pltpu.sync_copy(gbuf2.at[b], acc_sh.at[cbuf.at[j]], add=True)

    def group(g, carry):
        gg = g * _GR
        pltpu.sync_copy(row_hbm.at[pl.ds(e0 + gg, _GR)], rbuf)
        pltpu.sync_copy(col_hbm.at[pl.ds(e0 + gg, _GR)], cbuf)
        pltpu.sync_copy(ew_hbm.at[pl.ds(e0 + gg, _GR)], ebuf)
        start_gather(0, 0)

        def pair(j2, carry2):
            j0 = j2 * 2
            start_gather(j0 + 1, 1)
            chunk(j0, 0)

            @pl.when(j2 < _GR // 2 - 1)
            def _():
                start_gather(j0 + 2, 0)
            chunk(j0 + 1, 1)
            return carry2

        lax.fori_loop(0, _GR // 2, pair, 0)
        return carry

    lax.fori_loop(0, nch // _GR, group, 0)
    plsc.subcore_barrier()

    def finblk(t, carry):
        b0 = t * _FB
        pltpu.sync_copy(acc_sh.at[pl.ds(n0 + b0, _FB)], obuf)

        def fin(i, cc):
            sd = _splat16(dbuf, (b0 + i,))
            for f in range(nv):
                v = obuf[i, pl.ds(f * 16, 16)]
                b = bbuf[0, pl.ds(f * 16, 16)]
                v = v * sd + b
                if F == 64:
                    v = jnp.maximum(v, 0.0)
                obuf[i, pl.ds(f * 16, 16)] = v
            return cc

        lax.fori_loop(0, _FB, fin, 0)
        pltpu.sync_copy(obuf, out_hbm.at[c, pl.ds(n0 + b0, _FB)])
        return carry

    lax.fori_loop(0, nblk // _FB, finblk, 0)


_GR = 32
_FB = 64


def _spmm_kernel(F, rowp2, colp2, ewp2, Xs, dinv, bias):
    k = pl.kernel(
        functools.partial(_spmm_body, F),
        out_type=jax.ShapeDtypeStruct((2, NP, F), jnp.float32),
        mesh=_sc_mesh(),
        compiler_params=_SC_PARAMS,
        scratch_types=[
            pltpu.VMEM_SHARED((NP, F), jnp.float32),
            pltpu.VMEM_SHARED((NP, F), jnp.float32),
            pltpu.VMEM((_GR, 128), jnp.int32),
            pltpu.VMEM((_GR, 128), jnp.int32),
            pltpu.VMEM((_GR, 128), jnp.float32),
            pltpu.VMEM((2, 128, F), jnp.float32),
            pltpu.VMEM((_FB, F), jnp.float32),
            pltpu.VMEM((NP // _NS,), jnp.float32),
            pltpu.VMEM((1, F), jnp.float32),
            pltpu.SemaphoreType.DMA,
            pltpu.SemaphoreType.DMA,
        ],
    )
    return k(rowp2, colp2, ewp2, Xs, dinv, bias)


def kernel(x, edge_index, edge_x, W1, b1, W2, b2, Wc1, bc1, Wc2, bc2):
    xp = jnp.pad(x, ((0, NP - N), (0, 0)))
    rowp = jnp.pad(edge_index[0], (0, EP - E)).reshape(ECH, 128)
    colp = jnp.pad(edge_index[1], (0, EP - E)).reshape(ECH, 128)
    Wc2p = jnp.pad(Wc2, ((0, 0), (0, F2 - C)))
    bc2p = jnp.pad(bc2, (0, F2 - C))

    ew, X1 = _front(edge_x, W1, b1, W2, b2, xp, Wc1)
    ewp = jnp.pad(ew.reshape(-1), (0, EP - E)).reshape(ECH, 128)

    h1, dinv = _spmm1_kernel(rowp, colp, ewp, X1,
                             bc1.reshape(2, 1, 64))

    Xs2v = _mm2(h1, Wc2p, dinv.reshape(NP, 1))
    Xs2 = Xs2v.reshape(NP, 2, 32).transpose(1, 0, 2)

    out2 = _spmm_kernel(32, rowp, colp, ewp, Xs2, dinv,
                        bc2p.reshape(2, 1, 32))
    out = out2.transpose(1, 0, 2).reshape(NP, F2)
    return out[:N, :C]


# speedup vs baseline: 1.0441x; 1.0024x over previous
import functools

import jax
import jax.numpy as jnp
from jax import lax
from jax.experimental import pallas as pl
from jax.experimental.pallas import tpu as pltpu
from jax.experimental.pallas import tpu_sc as plsc

N = 10000
NP = 10240
E = 320000
EP = 327680
D = 128
F2 = 64
C = 40
ECH = EP // 128

_NC, _NS = 2, 16


def _splat16(val_ref, idxs):
    return plsc.load_gather(val_ref, [jnp.full((16,), i, jnp.int32) for i in idxs])


def _front_body(ex_ref, w1_ref, b1_ref, w2_ref, b2_ref, x_ref, wc1_ref,
                ew_ref, x1_ref):
    e = jnp.dot(ex_ref[...], w1_ref[...], preferred_element_type=jnp.float32)
    e = jnp.maximum(e + b1_ref[...], 0.0)
    o = jnp.dot(e, w2_ref[...], preferred_element_type=jnp.float32) + b2_ref[...]
    ew_ref[...] = jax.nn.sigmoid(o)
    r = jnp.dot(x_ref[...], wc1_ref[...], preferred_element_type=jnp.float32)
    x1_ref[0] = r[:, 0:64]
    x1_ref[1] = r[:, 64:128]


def _front(edge_x, W1, b1, W2, b2, xp, Wc1):
    be = E // 20
    bn = NP // 20
    return pl.pallas_call(
        _front_body,
        grid=(20,),
        in_specs=[
            pl.BlockSpec((be, 16), lambda i: (i, 0)),
            pl.BlockSpec((16, 16), lambda i: (0, 0)),
            pl.BlockSpec((1, 16), lambda i: (0, 0)),
            pl.BlockSpec((16, 1), lambda i: (0, 0)),
            pl.BlockSpec((1, 1), lambda i: (0, 0)),
            pl.BlockSpec((bn, D), lambda i: (i, 0)),
            pl.BlockSpec((D, D), lambda i: (0, 0)),
        ],
        out_specs=[
            pl.BlockSpec((be, 1), lambda i: (i, 0)),
            pl.BlockSpec((2, bn, 64), lambda i: (0, i, 0)),
        ],
        out_shape=[
            jax.ShapeDtypeStruct((E, 1), jnp.float32),
            jax.ShapeDtypeStruct((2, NP, 64), jnp.float32),
        ],
    )(edge_x, W1, b1.reshape(1, 16), W2, b2.reshape(1, 1), xp, Wc1)


def _mm2_body(ha_ref, hb_ref, w_ref, dinv_ref, xs_ref):
    x2 = (jnp.dot(ha_ref[0], w_ref[0:64, :], preferred_element_type=jnp.float32)
          + jnp.dot(hb_ref[0], w_ref[64:128, :],
                    preferred_element_type=jnp.float32))
    xs_ref[...] = x2 * dinv_ref[...]


def _mm2(h1, Wc2p, dinv):
    bn = 1024
    return pl.pallas_call(
        _mm2_body,
        grid=(NP // bn,),
        in_specs=[
            pl.BlockSpec((1, bn, 64), lambda i: (0, i, 0)),
            pl.BlockSpec((1, bn, 64), lambda i: (1, i, 0)),
            pl.BlockSpec((D, F2), lambda i: (0, 0)),
            pl.BlockSpec((bn, 1), lambda i: (i, 0)),
        ],
        out_specs=pl.BlockSpec((bn, F2), lambda i: (i, 0)),
        out_shape=jax.ShapeDtypeStruct((NP, F2), jnp.float32),
    )(h1, h1, Wc2p, dinv)


def _sc_mesh():
    return plsc.VectorSubcoreMesh(core_axis_name="c", subcore_axis_name="s")


_SC_PARAMS = pltpu.CompilerParams(needs_layout_passes=False,
                                  use_tc_tiling_on_sc=False)


def _spmm1_body(row_hbm, col_hbm, ew_hbm, x_hbm, b_hbm,
                out_hbm, dinv_out, deg_sh, xs_sh, acc_sh, rbuf, cbuf, ebuf,
                gbuf2, obuf, degbuf, dbuf, bbuf, xbuf, gsem0, gsem1):
    F = 64
    nv = F // 16
    c = lax.axis_index("c")
    s = lax.axis_index("s")
    n0 = s * (NP // _NS)
    nblk = NP // _NS
    nch = ECH // _NS
    e0 = s * nch

    def zset(k, carry):
        degbuf[pl.ds(k * 16, 16)] = jnp.zeros((16,), jnp.float32)
        return carry

    lax.fori_loop(0, nblk // 16, zset, 0)
    pltpu.sync_copy(degbuf, deg_sh.at[pl.ds(n0, nblk)])
    pltpu.sync_copy(b_hbm.at[c], bbuf)
    plsc.subcore_barrier()

    def dgroup(g, carry):
        gg = e0 + g * _GR
        pltpu.sync_copy(col_hbm.at[pl.ds(gg, _GR)], cbuf)
        pltpu.sync_copy(ew_hbm.at[pl.ds(gg, _GR)], ebuf)

        def dfire(j, carry2):
            pltpu.async_copy(ebuf.at[j], deg_sh.at[cbuf.at[j]], gsem0,
                             add=True)
            return carry2

        def ddrain(j, carry2):
            pltpu.make_async_copy(ebuf.at[j], deg_sh.at[cbuf.at[j]],
                                  gsem0).wait()
            return carry2

        lax.fori_loop(0, _GR, dfire, 0)
        lax.fori_loop(0, _GR, ddrain, 0)
        return carry

    lax.fori_loop(0, nch // _GR, dgroup, 0)
    plsc.subcore_barrier()

    pltpu.sync_copy(deg_sh.at[pl.ds(n0, nblk)], degbuf)

    def newton(k, carry):
        x = degbuf[pl.ds(k * 16, 16)] + 1.0
        i = plsc.bitcast(x, jnp.int32)
        i = 0x5F3759DF - lax.shift_right_logical(i, 1)
        y = plsc.bitcast(i, jnp.float32)
        for _ in range(3):
            y = y * (1.5 - 0.5 * x * y * y)
        dbuf[pl.ds(k * 16, 16)] = y
        return carry

    lax.fori_loop(0, nblk // 16, newton, 0)

    @pl.when(c == 0)
    def _():
        pltpu.sync_copy(dbuf, dinv_out.at[pl.ds(n0, nblk)])

    def xstage(t, carry):
        b0 = t * _XB
        pltpu.sync_copy(x_hbm.at[c, pl.ds(n0 + b0, _XB)], xbuf)

        def xscale(i, cc):
            sd = _splat16(dbuf, (b0 + i,))
            for f in range(nv):
                v = xbuf[i, pl.ds(f * 16, 16)]
                xbuf[i, pl.ds(f * 16, 16)] = v * sd
            return cc

        lax.fori_loop(0, _XB, xscale, 0)
        pltpu.sync_copy(xbuf, xs_sh.at[pl.ds(n0 + b0, _XB)])
        pltpu.sync_copy(xbuf, acc_sh.at[pl.ds(n0 + b0, _XB)])
        return carry

    lax.fori_loop(0, nblk // _XB, xstage, 0)
    plsc.subcore_barrier()

    gsems = (gsem0, gsem1)

    def start_gather(j, b):
        pltpu.async_copy(xs_sh.at[rbuf.at[j]], gbuf2.at[b], gsems[b])

    def wait_gather(j, b):
        pltpu.make_async_copy(xs_sh.at[rbuf.at[j]], gbuf2.at[b],
                              gsems[b]).wait()

    def chunk(j, b):
        wait_gather(j, b)

        def edge(i8, cc):
            for di in range(8):
                i = i8 * 8 + di
                sp = _splat16(ebuf, (j, i))
                for f in range(nv):
                    v = gbuf2[b, i, pl.ds(f * 16, 16)]
                    gbuf2[b, i, pl.ds(f * 16, 16)] = v * sp
            return cc

        lax.fori_loop(0, 16, edge, 0)
        pltpu.sync_copy(gbuf2.at[b], acc_sh.at[cbuf.at[j]], add=True)

    def group(g, carry):
        gg = g * _GR
        pltpu.sync_copy(row_hbm.at[pl.ds(e0 + gg, _GR)], rbuf)
        pltpu.sync_copy(col_hbm.at[pl.ds(e0 + gg, _GR)], cbuf)
        pltpu.sync_copy(ew_hbm.at[pl.ds(e0 + gg, _GR)], ebuf)
        start_gather(0, 0)

        def pair(j2, carry2):
            j0 = j2 * 2
            start_gather(j0 + 1, 1)
            chunk(j0, 0)

            @pl.when(j2 < _GR // 2 - 1)
            def _():
                start_gather(j0 + 2, 0)
            chunk(j0 + 1, 1)
            return carry2

        lax.fori_loop(0, _GR // 2, pair, 0)
        return carry

    lax.fori_loop(0, nch // _GR, group, 0)
    plsc.subcore_barrier()

    def finblk(t, carry):
        b0 = t * _FB
        pltpu.sync_copy(acc_sh.at[pl.ds(n0 + b0, _FB)], obuf)

        def fin(i, cc):
            sd = _splat16(dbuf, (b0 + i,))
            for f in range(nv):
                v = obuf[i, pl.ds(f * 16, 16)]
                b = bbuf[0, pl.ds(f * 16, 16)]
                v = jnp.maximum(v * sd + b, 0.0)
                obuf[i, pl.ds(f * 16, 16)] = v
            return cc

        lax.fori_loop(0, _FB, fin, 0)
        pltpu.sync_copy(obuf, out_hbm.at[c, pl.ds(n0 + b0, _FB)])
        return carry

    lax.fori_loop(0, nblk // _FB, finblk, 0)


_XB = 160


def _spmm1_kernel(rowp2, colp2, ewp2, X1, bias):
    k = pl.kernel(
        _spmm1_body,
        out_type=[jax.ShapeDtypeStruct((2, NP, 64), jnp.float32),
                  jax.ShapeDtypeStruct((NP,), jnp.float32)],
        mesh=_sc_mesh(),
        compiler_params=_SC_PARAMS,
        scratch_types=[
            pltpu.VMEM_SHARED((NP,), jnp.float32),
            pltpu.VMEM_SHARED((NP, 64), jnp.float32),
            pltpu.VMEM_SHARED((NP, 64), jnp.float32),
            pltpu.VMEM((_GR, 128), jnp.int32),
            pltpu.VMEM((_GR, 128), jnp.int32),
            pltpu.VMEM((_GR, 128), jnp.float32),
            pltpu.VMEM((2, 128, 64), jnp.float32),
            pltpu.VMEM((_FB, 64), jnp.float32),
            pltpu.VMEM((NP // _NS,), jnp.float32),
            pltpu.VMEM((NP // _NS,), jnp.float32),
            pltpu.VMEM((1, 64), jnp.float32),
            pltpu.VMEM((_XB, 64), jnp.float32),
            pltpu.SemaphoreType.DMA,
            pltpu.SemaphoreType.DMA,
        ],
    )
    return k(rowp2, colp2, ewp2, X1, bias)


def _spmm_body(F, row_hbm, col_hbm, ew_hbm, xs_hbm, dinv_hbm, b_hbm,
               out_hbm, xs_sh, acc_sh, rbuf, cbuf, ebuf, gbuf2, obuf,
               dbuf, bbuf, gsem0, gsem1):
    nv = F // 16
    c = lax.axis_index("c")
    s = lax.axis_index("s")
    n0 = s * (NP // _NS)
    nblk = NP // _NS
    pltpu.sync_copy(xs_hbm.at[c, pl.ds(n0, nblk)], xs_sh.at[pl.ds(n0, nblk)])
    pltpu.sync_copy(xs_hbm.at[c, pl.ds(n0, nblk)], acc_sh.at[pl.ds(n0, nblk)])
    pltpu.sync_copy(dinv_hbm.at[pl.ds(n0, nblk)], dbuf)
    pltpu.sync_copy(b_hbm.at[c], bbuf)
    plsc.subcore_barrier()

    nch = ECH // _NS
    e0 = s * nch
    gsems = (gsem0, gsem1)

    def start_gather(j, b):
        pltpu.async_copy(xs_sh.at[rbuf.at[j]], gbuf2.at[b], gsems[b])

    def wait_gather(j, b):
        pltpu.make_async_copy(xs_sh.at[rbuf.at[j]], gbuf2.at[b],
                              gsems[b]).wait()

    def chunk(j, b):
        wait_gather(j, b)

        def edge(i8, cc):
            for di in range(8):
                i = i8 * 8 + di
                sp = _splat16(ebuf, (j, i))
                for f in range(nv):
                    v = gbuf2[b, i, pl.ds(f * 16, 16)]
                    gbuf2[b, i, pl.ds(f * 16, 16)] = v * sp
            return cc

        lax.fori_loop(0, 16, edge, 0)
        pltpu.sync_copy(gbuf2.at[b], acc_sh.at[cbuf.at[j]], add=True)

    def group(g, carry):
        gg = g * _GR
        pltpu.sync_copy(row_hbm.at[pl.ds(e0 + gg, _GR)], rbuf)
        pltpu.sync_copy(col_hbm.at[pl.ds(e0 + gg, _GR)], cbuf)
        pltpu.sync_copy(ew_hbm.at[pl.ds(e0 + gg, _GR)], ebuf)
        start_gather(0, 0)

        def pair(j2, carry2):
            j0 = j2 * 2
            start_gather(j0 + 1, 1)
            chunk(j0, 0)

            @pl.when(j2 < _GR // 2 - 1)
            def _():
                start_gather(j0 + 2, 0)
            chunk(j0 + 1, 1)
            return carry2

        lax.fori_loop(0, _GR // 2, pair, 0)
        return carry

    lax.fori_loop(0, nch // _GR, group, 0)
    plsc.subcore_barrier()

    def finblk(t, carry):
        b0 = t * _FB
        pltpu.sync_copy(acc_sh.at[pl.ds(n0 + b0, _FB)], obuf)

        def fin(i, cc):
            sd = _splat16(dbuf, (b0 + i,))
            for f in range(nv):
                v = obuf[i, pl.ds(f * 16, 16)]
                b = bbuf[0, pl.ds(f * 16, 16)]
                v = v * sd + b
                if F == 64:
                    v = jnp.maximum(v, 0.0)
                obuf[i, pl.ds(f * 16, 16)] = v
            return cc

        lax.fori_loop(0, _FB, fin, 0)
        pltpu.sync_copy(obuf, out_hbm.at[c, pl.ds(n0 + b0, _FB)])
        return carry

    lax.fori_loop(0, nblk // _FB, finblk, 0)


_GR = 32
_FB = 64


def _spmm_kernel(F, rowp2, colp2, ewp2, Xs, dinv, bias):
    k = pl.kernel(
        functools.partial(_spmm_body, F),
        out_type=jax.ShapeDtypeStruct((2, NP, F), jnp.float32),
        mesh=_sc_mesh(),
        compiler_params=_SC_PARAMS,
        scratch_types=[
            pltpu.VMEM_SHARED((NP, F), jnp.float32),
            pltpu.VMEM_SHARED((NP, F), jnp.float32),
            pltpu.VMEM((_GR, 128), jnp.int32),
            pltpu.VMEM((_GR, 128), jnp.int32),
            pltpu.VMEM((_GR, 128), jnp.float32),
            pltpu.VMEM((2, 128, F), jnp.float32),
            pltpu.VMEM((_FB, F), jnp.float32),
            pltpu.VMEM((NP // _NS,), jnp.float32),
            pltpu.VMEM((1, F), jnp.float32),
            pltpu.SemaphoreType.DMA,
            pltpu.SemaphoreType.DMA,
        ],
    )
    return k(rowp2, colp2, ewp2, Xs, dinv, bias)


def kernel(x, edge_index, edge_x, W1, b1, W2, b2, Wc1, bc1, Wc2, bc2):
    xp = jnp.pad(x, ((0, NP - N), (0, 0)))
    rowp = jnp.pad(edge_index[0], (0, EP - E)).reshape(ECH, 128)
    colp = jnp.pad(edge_index[1], (0, EP - E)).reshape(ECH, 128)
    Wc2p = jnp.pad(Wc2, ((0, 0), (0, F2 - C)))
    bc2p = jnp.pad(bc2, (0, F2 - C))

    ew, X1 = _front(edge_x, W1, b1, W2, b2, xp, Wc1)
    ewp = jnp.pad(ew.reshape(-1), (0, EP - E)).reshape(ECH, 128)

    h1, dinv = _spmm1_kernel(rowp, colp, ewp, X1,
                             bc1.reshape(2, 1, 64))

    Xs2v = _mm2(h1, Wc2p, dinv.reshape(NP, 1))
    Xs2 = Xs2v.reshape(NP, 2, 32).transpose(1, 0, 2)

    out2 = _spmm_kernel(32, rowp, colp, ewp, Xs2, dinv,
                        bc2p.reshape(2, 1, 32))
    out = out2.transpose(1, 0, 2).reshape(NP, F2)
    return out[:N, :C]
